# Initial kernel scaffold; baseline (speedup 1.0000x reference)
import jax
import jax.numpy as jnp
from jax.experimental import pallas as pl


def kernel(x, table):
    raise NotImplementedError("write your pallas kernel here")


# speedup vs baseline: 4.0818x; 4.0818x over previous
import functools

import jax
import jax.numpy as jnp
from jax import lax
from jax.experimental import pallas as pl
from jax.experimental.pallas import tpu as pltpu
from jax.experimental.pallas import tpu_sc as plsc

NUM_CORES = 2
NUM_SUBCORES = 16
NW = NUM_CORES * NUM_SUBCORES

B = 4096 * 50
D = 64
B_PER_W = B // NW
CH = 128
N_CH = B_PER_W // CH


def _emb_kernel(x_hbm, table_hbm, out_hbm, idx_v, rows_v, in_sem, out_sem):
    wid = lax.axis_index("s") * NUM_CORES + lax.axis_index("c")
    base = wid * B_PER_W
    pltpu.sync_copy(x_hbm.at[pl.ds(base, B_PER_W)], idx_v)

    @pl.loop(0, N_CH)
    def _chunk(j):
        pltpu.async_copy(
            table_hbm.at[idx_v.at[pl.ds(j * CH, CH)]],
            rows_v,
            in_sem,
        ).wait()
        pltpu.async_copy(
            rows_v,
            out_hbm.at[pl.ds(base + j * CH, CH)],
            out_sem,
        ).wait()


@jax.jit
def _emb(x_flat, table):
    run = pl.kernel(
        _emb_kernel,
        out_type=jax.ShapeDtypeStruct((B, D), jnp.float32),
        mesh=plsc.VectorSubcoreMesh(
            core_axis_name="c", subcore_axis_name="s"
        ),
        scratch_types=[
            pltpu.VMEM((B_PER_W,), jnp.int32),
            pltpu.VMEM((CH, D), jnp.float32),
            pltpu.SemaphoreType.DMA,
            pltpu.SemaphoreType.DMA,
        ],
        compiler_params=pltpu.CompilerParams(use_tc_tiling_on_sc=False),
    )
    return run(x_flat, table)


def kernel(x, table):
    out = _emb(x.reshape(-1).astype(jnp.int32), table)
    return out.reshape(x.shape + (D,))


# speedup vs baseline: 4.2666x; 1.0453x over previous
import jax
import jax.numpy as jnp
from jax import lax
from jax.experimental import pallas as pl
from jax.experimental.pallas import tpu as pltpu
from jax.experimental.pallas import tpu_sc as plsc

NUM_CORES = 2
NUM_SUBCORES = 16
NW = NUM_CORES * NUM_SUBCORES

B = 4096 * 50
D = 64
B_PER_W = B // NW
CH = 128
N_CH = B_PER_W // CH


def _emb_kernel(x_hbm, table_hbm, out_hbm, idx_v, rows0, rows1,
                g0, g1, o0, o1):
    wid = lax.axis_index("s") * NUM_CORES + lax.axis_index("c")
    base = wid * B_PER_W
    pltpu.sync_copy(x_hbm.at[pl.ds(base, B_PER_W)], idx_v)

    bufs = (rows0, rows1)
    gsems = (g0, g1)
    osems = (o0, o1)

    def gather(j, b):
        return pltpu.make_async_copy(
            table_hbm.at[idx_v.at[pl.ds(j * CH, CH)]], bufs[b], gsems[b])

    def write(j, b):
        return pltpu.make_async_copy(
            bufs[b], out_hbm.at[pl.ds(base + j * CH, CH)], osems[b])

    gather(0, 0).start()
    gather(0, 0).wait()
    gather(1, 1).start()
    write(0, 0).start()

    gather(1, 1).wait()
    write(0, 0).wait()
    gather(2, 0).start()
    write(1, 1).start()

    @pl.loop(1, N_CH // 2 - 1)
    def _grp(g):
        j0 = 2 * g
        gather(j0, 0).wait()
        write(j0 - 1, 1).wait()
        gather(j0 + 1, 1).start()
        write(j0, 0).start()

        gather(j0 + 1, 1).wait()
        write(j0, 0).wait()
        gather(j0 + 2, 0).start()
        write(j0 + 1, 1).start()

    jl = N_CH - 2
    gather(jl, 0).wait()
    write(jl - 1, 1).wait()
    gather(jl + 1, 1).start()
    write(jl, 0).start()

    gather(jl + 1, 1).wait()
    write(jl, 0).wait()
    write(jl + 1, 1).start()
    write(jl + 1, 1).wait()


@jax.jit
def _emb(x_flat, table):
    run = pl.kernel(
        _emb_kernel,
        out_type=jax.ShapeDtypeStruct((B, D), jnp.float32),
        mesh=plsc.VectorSubcoreMesh(
            core_axis_name="c", subcore_axis_name="s"
        ),
        scratch_types=[
            pltpu.VMEM((B_PER_W,), jnp.int32),
            pltpu.VMEM((CH, D), jnp.float32),
            pltpu.VMEM((CH, D), jnp.float32),
            pltpu.SemaphoreType.DMA,
            pltpu.SemaphoreType.DMA,
            pltpu.SemaphoreType.DMA,
            pltpu.SemaphoreType.DMA,
        ],
        compiler_params=pltpu.CompilerParams(use_tc_tiling_on_sc=False),
    )
    return run(x_flat, table)


def kernel(x, table):
    out = _emb(x.reshape(-1).astype(jnp.int32), table)
    return out.reshape(x.shape + (D,))


# speedup vs baseline: 4.6754x; 1.0958x over previous
import jax
import jax.numpy as jnp
from jax import lax
from jax.experimental import pallas as pl
from jax.experimental.pallas import tpu as pltpu
from jax.experimental.pallas import tpu_sc as plsc

NUM_CORES = 2
NUM_SUBCORES = 16
NW = NUM_CORES * NUM_SUBCORES

B = 4096 * 50
D = 64
B_PER_W = B // NW
CH = 128
N_CH = B_PER_W // CH
NB = 4


def _emb_kernel(x_hbm, table_hbm, out_hbm, idx_v, *bufs_and_sems):
    bufs = bufs_and_sems[:NB]
    gsems = bufs_and_sems[NB:2 * NB]
    osems = bufs_and_sems[2 * NB:3 * NB]

    wid = lax.axis_index("s") * NUM_CORES + lax.axis_index("c")
    base = wid * B_PER_W
    pltpu.sync_copy(x_hbm.at[pl.ds(base, B_PER_W)], idx_v)

    def gather(j, b):
        return pltpu.make_async_copy(
            table_hbm.at[idx_v.at[pl.ds(j * CH, CH)]], bufs[b], gsems[b])

    def write(j, b):
        return pltpu.make_async_copy(
            bufs[b], out_hbm.at[pl.ds(base + j * CH, CH)], osems[b])

    def step(j, b, fire, wait_prev_write):
        pb = (b - 1) % NB
        gather(j, b).wait()
        if fire:
            if wait_prev_write:
                write(j - 1, pb).wait()
            gather(j + NB - 1, pb).start()
        write(j, b).start()

    for b in range(NB - 1):
        gather(b, b).start()

    step(0, 0, fire=True, wait_prev_write=False)

    last_fire = N_CH - NB
    n_grp = last_fire // NB
    grp_end = 1 + n_grp * NB

    @pl.loop(0, n_grp)
    def _grp(g):
        j0 = NB * g + 1
        for t in range(NB):
            step(j0 + t, (1 + t) % NB, fire=True, wait_prev_write=True)

    for j in range(grp_end, last_fire + 1):
        step(j, j % NB, fire=True, wait_prev_write=True)
    for j in range(last_fire + 1, N_CH):
        step(j, j % NB, fire=False, wait_prev_write=False)

    for j in range(N_CH - NB, N_CH):
        write(j, j % NB).wait()


@jax.jit
def _emb(x_flat, table):
    run = pl.kernel(
        _emb_kernel,
        out_type=jax.ShapeDtypeStruct((B, D), jnp.float32),
        mesh=plsc.VectorSubcoreMesh(
            core_axis_name="c", subcore_axis_name="s"
        ),
        scratch_types=(
            [pltpu.VMEM((B_PER_W,), jnp.int32)]
            + [pltpu.VMEM((CH, D), jnp.float32)] * NB
            + [pltpu.SemaphoreType.DMA] * (2 * NB)
        ),
        compiler_params=pltpu.CompilerParams(use_tc_tiling_on_sc=False),
    )
    return run(x_flat, table)


def kernel(x, table):
    out = _emb(x.reshape(-1).astype(jnp.int32), table)
    return out.reshape(x.shape + (D,))
